# baseline (speedup 1.0000x reference)
import numpy as np
import jax
import jax.numpy as jnp
from jax import lax
from jax.experimental import pallas as pl
from jax.experimental.pallas import tpu as pltpu
from jax.experimental.pallas import tpu_sc as plsc

_P = 3
_D = 27
_DP = 32
_B = 4
_NQ = 256
_G = 1468
_GP = 1472
_NGRP = _GP // 16
_NW = 32
_QPW = (_B * _NQ) // _NW

_CV = 3 * 48 * 48
_NSRC = 3 * _CV


def _cubic(t):
    a = -0.75
    at = jnp.abs(t)
    w1 = ((a + 2.0) * at - (a + 3.0)) * at * at + 1.0
    w2 = (((at - 5.0) * at + 8.0) * at - 4.0) * a
    return jnp.where(at <= 1.0, w1, jnp.where(at < 2.0, w2, 0.0))


def _resize1d(x, out_size, axis):
    in_size = x.shape[axis]
    o = jnp.arange(out_size, dtype=jnp.float32)
    src = o * ((in_size - 1) / (out_size - 1))
    i0 = jnp.floor(src).astype(jnp.int32)
    ts = src - i0.astype(jnp.float32)
    offs = jnp.arange(-1, 3)
    idx = jnp.clip(i0[:, None] + offs[None, :], 0, in_size - 1)
    w = _cubic(ts[:, None] - offs[None, :].astype(jnp.float32))
    xm = jnp.moveaxis(x, axis, -1)
    g = xm[..., idx]
    res = jnp.sum(g * w, axis=-1)
    return jnp.moveaxis(res, -1, axis)


def _resize(x, scale):
    _, _, h, w = x.shape
    x = _resize1d(x, int(round(h * scale)), 2)
    x = _resize1d(x, int(round(w * scale)), 3)
    return x


def _base_addrs():
    sizes = [(0, 48), (_CV, 24), (2 * _CV, 12)]
    rows = []

    def piece(base, s, i, j):
        n = (s - _P) // _P if (i or j) else s // _P
        for bi in range(n):
            for bj in range(n):
                rows.append(base + (i + bi * _P) * 48 + j + bj * _P)

    for i in range(1, _P):
        for j in range(1, _P):
            for base, s in sizes:
                piece(base, s, i, j)
    for base, s in sizes:
        piece(base, s, 0, 0)
    assert len(rows) == _G
    rows.extend([2 * _CV + 45 * 48 + 45] * (_GP - _G))
    return np.asarray(rows, np.int32)


_OFF = [c * 2304 + pi * 48 + pj
        for c in range(3) for pi in range(_P) for pj in range(_P)]
_BA = _base_addrs()


def _pad16(v):
    v = list(v)
    return [v[:16], v[16:] + [v[26]] * (_DP - _D)]


_OFFS = np.asarray(
    _pad16(_OFF)
    + _pad16([d // 9 for d in range(_D)])
    + _pad16([(d % 9) // 3 for d in range(_D)])
    + _pad16([d % 3 for d in range(_D)]),
    np.int32)


def _sc_nn_kernel(src_hbm, inp_hbm, ba_hbm, offs_hbm,
                  sel_hbm, loss_hbm,
                  src_v, inp_v, ba_v, offs_v,
                  gc_v, t_v, i_v, img_v, loss_v):
    nc = 2
    wid = lax.axis_index("s") * nc + lax.axis_index("c")
    b = wid // (_NQ // _QPW)
    qbase = (wid % (_NQ // _QPW)) * _QPW

    pltpu.sync_copy(src_hbm.at[b], src_v)
    pltpu.sync_copy(inp_hbm.at[b], inp_v)
    pltpu.sync_copy(ba_hbm, ba_v)
    pltpu.sync_copy(offs_hbm, offs_v)

    lanes = lax.iota(jnp.int32, 16)
    dmask = lanes < (_D - 16)
    orows = [offs_v[k, :] for k in range(8)]

    def stage_q(qi, c):
        q = qbase + qi
        av = lax.broadcast(144 * (q // 16) + 3 * (q % 16), (16,))
        ia = av + orows[0]
        ib = av + orows[1]
        t_v[qi, pl.ds(0, 16)] = plsc.load_gather(src_v, [ia])
        t_v[qi, pl.ds(16, 16)] = plsc.load_gather(src_v, [ib])
        i_v[qi, pl.ds(0, 16)] = plsc.load_gather(inp_v, [ia])
        i_v[qi, pl.ds(16, 16)] = jnp.where(
            dmask, plsc.load_gather(inp_v, [ib]), 0.0)
        return c

    lax.fori_loop(0, _QPW, stage_q, 0)

    def stage_g(gi, c):
        base = gi * 16
        av = ba_v[pl.ds(base, 16)]
        for d in range(_D):
            gc_v[d, pl.ds(base, 16)] = plsc.load_gather(src_v, [av + _OFF[d]])
        return c

    lax.fori_loop(0, _NGRP, stage_g, 0)

    def msp_of(qi):
        t_a = t_v[qi, pl.ds(0, 16)]
        t_b = t_v[qi, pl.ds(16, 16)]
        i_a = i_v[qi, pl.ds(0, 16)]
        i_b = i_v[qi, pl.ds(16, 16)]
        m_a = (t_a + i_a) * 0.5
        m_b = (t_b + i_b) * 0.5
        msp = []
        for d in range(_D):
            ms = m_a[d] if d < 16 else m_b[d - 16]
            msp.append(lax.broadcast(ms, (16,)))
        return msp, i_a, i_b

    def select_q(qi, cstar, i_a, i_b, lossacc):
        av = plsc.load_gather(ba_v, [lax.broadcast(cstar, (16,))])
        sela = plsc.load_gather(src_v, [av + orows[0]])
        selb = jnp.where(dmask, plsc.load_gather(src_v, [av + orows[1]]), 0.0)
        rowb = lax.broadcast(3 * (qi // 16), (16,))
        colb = lax.broadcast(3 * (qi % 16), (16,))
        plsc.store_scatter(
            img_v, [orows[2], rowb + orows[4], colb + orows[6]], sela)
        plsc.store_scatter(
            img_v, [orows[3], rowb + orows[5], colb + orows[7]],
            selb, mask=dmask)
        return (lossacc + jnp.abs(i_a - sela)) + jnp.abs(i_b - selb)

    def q_body(qp, lossacc):
        qs = [4 * qp + k for k in range(4)]
        mspq = []
        iaq = []
        for q in qs:
            msp, ia, ib = msp_of(q)
            mspq.append(msp)
            iaq.append((ia, ib))

        def g_body(gi, carry):
            mvs = list(carry[0:4])
            mgs = list(carry[4:8])
            accs = [[jnp.zeros((16,), jnp.float32) for _ in range(4)]
                    for _ in range(4)]
            base = gi * 16
            for d in range(_D):
                gv = gc_v[d, pl.ds(base, 16)]
                for k in range(4):
                    df = mspq[k][d] - gv
                    accs[k][d % 4] = accs[k][d % 4] + df * df
            gb = lax.broadcast(gi, (16,))
            for k in range(4):
                a = accs[k]
                s = (a[0] + a[1]) + (a[2] + a[3])
                bt = s < mvs[k]
                mvs[k] = jnp.where(bt, s, mvs[k])
                mgs[k] = jnp.where(bt, gb, mgs[k])
            return tuple(mvs) + tuple(mgs)

        mv0 = jnp.full((16,), 3.0e38, jnp.float32)
        mg0 = jnp.zeros((16,), jnp.int32)
        res = lax.fori_loop(0, _NGRP, g_body, (mv0,) * 4 + (mg0,) * 4)

        def argmin_of(mv, mg):
            gmin = jnp.min(mv)
            cand = mg * 16 + lanes
            masked = jnp.where(mv == gmin, cand, jnp.int32(2**30))
            return jnp.min(masked)

        for k in range(4):
            lossacc = select_q(qs[k], argmin_of(res[k], res[4 + k]),
                               iaq[k][0], iaq[k][1], lossacc)
        return lossacc

    lossacc = lax.fori_loop(0, _QPW // 4, q_body,
                            jnp.zeros((16,), jnp.float32))
    loss_v[...] = lossacc
    pltpu.sync_copy(img_v, sel_hbm.at[b, wid % (_NQ // _QPW)])
    pltpu.sync_copy(loss_v, loss_hbm.at[wid])


def kernel(inp, tar):
    x2 = jnp.pad(_resize(tar, 0.5), ((0, 0), (0, 0), (0, 24), (0, 24)),
                 constant_values=1e9)
    x4 = jnp.pad(_resize(tar, 0.25), ((0, 0), (0, 0), (0, 36), (0, 36)),
                 constant_values=1e9)
    src = jnp.concatenate(
        [tar.reshape(_B, -1), x2.reshape(_B, -1), x4.reshape(_B, -1)],
        axis=1)
    inpf = inp.reshape(_B, -1)

    mesh = plsc.VectorSubcoreMesh(core_axis_name="c", subcore_axis_name="s")
    sel, lossp = pl.kernel(
        _sc_nn_kernel,
        mesh=mesh,
        compiler_params=pltpu.CompilerParams(
            needs_layout_passes=False, use_tc_tiling_on_sc=False),
        out_type=[
            jax.ShapeDtypeStruct((_B, 8, 3, 6, 48), jnp.float32),
            jax.ShapeDtypeStruct((_NW, 16), jnp.float32),
        ],
        scratch_types=[
            pltpu.VMEM((_NSRC,), jnp.float32),
            pltpu.VMEM((_CV,), jnp.float32),
            pltpu.VMEM((_GP,), jnp.int32),
            pltpu.VMEM((8, 16), jnp.int32),
            pltpu.VMEM((_D, _GP), jnp.float32),
            pltpu.VMEM((_QPW, _DP), jnp.float32),
            pltpu.VMEM((_QPW, _DP), jnp.float32),
            pltpu.VMEM((3, 6, 48), jnp.float32),
            pltpu.VMEM((16,), jnp.float32),
        ],
    )(src, inpf, jnp.asarray(_BA), jnp.asarray(_OFFS))

    sel_img = jnp.transpose(sel, (0, 2, 1, 3, 4)).reshape(_B, 3, 48, 48)
    loss = lossp.sum() / (_B * _NQ * _D)
    return loss, sel_img


# speedup vs baseline: 1.0389x; 1.0389x over previous
import numpy as np
import jax
import jax.numpy as jnp
from jax import lax
from jax.experimental import pallas as pl
from jax.experimental.pallas import tpu as pltpu
from jax.experimental.pallas import tpu_sc as plsc

_P = 3
_D = 27
_DP = 32
_B = 4
_NQ = 256
_G = 1468
_GP = 1472
_NGRP = _GP // 16
_NW = 32
_QPW = (_B * _NQ) // _NW

_CV = 3 * 48 * 48
_NSRC = 3 * _CV


def _cubic(t):
    a = -0.75
    at = jnp.abs(t)
    w1 = ((a + 2.0) * at - (a + 3.0)) * at * at + 1.0
    w2 = (((at - 5.0) * at + 8.0) * at - 4.0) * a
    return jnp.where(at <= 1.0, w1, jnp.where(at < 2.0, w2, 0.0))


def _resize1d(x, out_size, axis):
    in_size = x.shape[axis]
    o = jnp.arange(out_size, dtype=jnp.float32)
    src = o * ((in_size - 1) / (out_size - 1))
    i0 = jnp.floor(src).astype(jnp.int32)
    ts = src - i0.astype(jnp.float32)
    offs = jnp.arange(-1, 3)
    idx = jnp.clip(i0[:, None] + offs[None, :], 0, in_size - 1)
    w = _cubic(ts[:, None] - offs[None, :].astype(jnp.float32))
    xm = jnp.moveaxis(x, axis, -1)
    g = xm[..., idx]
    res = jnp.sum(g * w, axis=-1)
    return jnp.moveaxis(res, -1, axis)


def _resize(x, scale):
    _, _, h, w = x.shape
    x = _resize1d(x, int(round(h * scale)), 2)
    x = _resize1d(x, int(round(w * scale)), 3)
    return x


def _base_addrs():
    sizes = [(0, 48), (_CV, 24), (2 * _CV, 12)]
    rows = []

    def piece(base, s, i, j):
        n = (s - _P) // _P if (i or j) else s // _P
        for bi in range(n):
            for bj in range(n):
                rows.append(base + (i + bi * _P) * 48 + j + bj * _P)

    for i in range(1, _P):
        for j in range(1, _P):
            for base, s in sizes:
                piece(base, s, i, j)
    for base, s in sizes:
        piece(base, s, 0, 0)
    assert len(rows) == _G
    rows.extend([2 * _CV + 45 * 48 + 45] * (_GP - _G))
    return np.asarray(rows, np.int32)


_OFF = [c * 2304 + pi * 48 + pj
        for c in range(3) for pi in range(_P) for pj in range(_P)]
_BA = _base_addrs()


def _pad16(v):
    v = list(v)
    return [v[:16], v[16:] + [v[26]] * (_DP - _D)]


_OFFS = np.asarray(
    _pad16(_OFF)
    + _pad16([d // 9 for d in range(_D)])
    + _pad16([(d % 9) // 3 for d in range(_D)])
    + _pad16([d % 3 for d in range(_D)]),
    np.int32)


def _sc_nn_kernel(src_hbm, inp_hbm, ba_hbm, offs_hbm,
                  sel_hbm, loss_hbm,
                  src_v, inp_v, ba_v, offs_v,
                  gc_v, t_v, i_v, img_v, loss_v):
    nc = 2
    wid = lax.axis_index("s") * nc + lax.axis_index("c")
    b = wid // (_NQ // _QPW)
    qbase = (wid % (_NQ // _QPW)) * _QPW

    pltpu.sync_copy(src_hbm.at[b], src_v)
    pltpu.sync_copy(inp_hbm.at[b], inp_v)
    pltpu.sync_copy(ba_hbm, ba_v)
    pltpu.sync_copy(offs_hbm, offs_v)

    lanes = lax.iota(jnp.int32, 16)
    dmask = lanes < (_D - 16)
    orows = [offs_v[k, :] for k in range(8)]

    def stage_q(qi, c):
        q = qbase + qi
        av = lax.broadcast(144 * (q // 16) + 3 * (q % 16), (16,))
        ia = av + orows[0]
        ib = av + orows[1]
        t_v[qi, pl.ds(0, 16)] = plsc.load_gather(src_v, [ia])
        t_v[qi, pl.ds(16, 16)] = plsc.load_gather(src_v, [ib])
        i_v[qi, pl.ds(0, 16)] = plsc.load_gather(inp_v, [ia])
        i_v[qi, pl.ds(16, 16)] = jnp.where(
            dmask, plsc.load_gather(inp_v, [ib]), 0.0)
        return c

    lax.fori_loop(0, _QPW, stage_q, 0)

    def stage_g(gi, c):
        base = gi * 16
        av = ba_v[pl.ds(base, 16)]
        for d in range(_D):
            gc_v[d, pl.ds(base, 16)] = plsc.load_gather(src_v, [av + _OFF[d]])
        return c

    lax.fori_loop(0, _NGRP, stage_g, 0)

    def msp_of(qi):
        t_a = t_v[qi, pl.ds(0, 16)]
        t_b = t_v[qi, pl.ds(16, 16)]
        i_a = i_v[qi, pl.ds(0, 16)]
        i_b = i_v[qi, pl.ds(16, 16)]
        m_a = (t_a + i_a) * 0.5
        m_b = (t_b + i_b) * 0.5
        msp = []
        for d in range(_D):
            ms = m_a[d] if d < 16 else m_b[d - 16]
            msp.append(lax.broadcast(ms, (16,)))
        return msp, i_a, i_b

    def select_q(qi, cstar, i_a, i_b, lossacc):
        av = plsc.load_gather(ba_v, [lax.broadcast(cstar, (16,))])
        sela = plsc.load_gather(src_v, [av + orows[0]])
        selb = jnp.where(dmask, plsc.load_gather(src_v, [av + orows[1]]), 0.0)
        rowb = lax.broadcast(3 * (qi // 16), (16,))
        colb = lax.broadcast(3 * (qi % 16), (16,))
        plsc.store_scatter(
            img_v, [orows[2], rowb + orows[4], colb + orows[6]], sela)
        plsc.store_scatter(
            img_v, [orows[3], rowb + orows[5], colb + orows[7]],
            selb, mask=dmask)
        return (lossacc + jnp.abs(i_a - sela)) + jnp.abs(i_b - selb)

    def q_body(qp, lossacc):
        q1 = 2 * qp
        q2 = q1 + 1
        msp1, ia1, ib1 = msp_of(q1)
        msp2, ia2, ib2 = msp_of(q2)

        def g_body(gi, carry):
            mv1, mg1, mv2, mg2 = carry
            a1 = [jnp.zeros((16,), jnp.float32) for _ in range(4)]
            a2 = [jnp.zeros((16,), jnp.float32) for _ in range(4)]
            base = gi * 16
            for d in range(_D):
                gv = gc_v[d, pl.ds(base, 16)]
                df1 = msp1[d] - gv
                a1[d % 4] = a1[d % 4] + df1 * df1
                df2 = msp2[d] - gv
                a2[d % 4] = a2[d % 4] + df2 * df2
            s1 = (a1[0] + a1[1]) + (a1[2] + a1[3])
            s2 = (a2[0] + a2[1]) + (a2[2] + a2[3])
            gb = lax.broadcast(gi, (16,))
            b1 = s1 < mv1
            mv1 = jnp.where(b1, s1, mv1)
            mg1 = jnp.where(b1, gb, mg1)
            b2 = s2 < mv2
            mv2 = jnp.where(b2, s2, mv2)
            mg2 = jnp.where(b2, gb, mg2)
            return mv1, mg1, mv2, mg2

        mv0 = jnp.full((16,), 3.0e38, jnp.float32)
        mg0 = jnp.zeros((16,), jnp.int32)
        mv1, mg1, mv2, mg2 = lax.fori_loop(
            0, _NGRP, g_body, (mv0, mg0, mv0, mg0))

        def argmin_of(mv, mg):
            gmin = jnp.min(mv)
            cand = mg * 16 + lanes
            masked = jnp.where(mv == gmin, cand, jnp.int32(2**30))
            return jnp.min(masked)

        lossacc = select_q(q1, argmin_of(mv1, mg1), ia1, ib1, lossacc)
        return select_q(q2, argmin_of(mv2, mg2), ia2, ib2, lossacc)

    lossacc = lax.fori_loop(0, _QPW // 2, q_body,
                            jnp.zeros((16,), jnp.float32))
    loss_v[...] = lossacc
    pltpu.sync_copy(img_v, sel_hbm.at[b, wid % (_NQ // _QPW)])
    pltpu.sync_copy(loss_v, loss_hbm.at[wid])


def kernel(inp, tar):
    x2 = jnp.pad(_resize(tar, 0.5), ((0, 0), (0, 0), (0, 24), (0, 24)),
                 constant_values=1e9)
    x4 = jnp.pad(_resize(tar, 0.25), ((0, 0), (0, 0), (0, 36), (0, 36)),
                 constant_values=1e9)
    src = jnp.concatenate(
        [tar.reshape(_B, -1), x2.reshape(_B, -1), x4.reshape(_B, -1)],
        axis=1)
    inpf = inp.reshape(_B, -1)

    mesh = plsc.VectorSubcoreMesh(core_axis_name="c", subcore_axis_name="s")
    sel, lossp = pl.kernel(
        _sc_nn_kernel,
        mesh=mesh,
        compiler_params=pltpu.CompilerParams(
            needs_layout_passes=False, use_tc_tiling_on_sc=False),
        out_type=[
            jax.ShapeDtypeStruct((_B, 8, 3, 6, 48), jnp.float32),
            jax.ShapeDtypeStruct((_NW, 16), jnp.float32),
        ],
        scratch_types=[
            pltpu.VMEM((_NSRC,), jnp.float32),
            pltpu.VMEM((_CV,), jnp.float32),
            pltpu.VMEM((_GP,), jnp.int32),
            pltpu.VMEM((8, 16), jnp.int32),
            pltpu.VMEM((_D, _GP), jnp.float32),
            pltpu.VMEM((_QPW, _DP), jnp.float32),
            pltpu.VMEM((_QPW, _DP), jnp.float32),
            pltpu.VMEM((3, 6, 48), jnp.float32),
            pltpu.VMEM((16,), jnp.float32),
        ],
    )(src, inpf, jnp.asarray(_BA), jnp.asarray(_OFFS))

    sel_img = jnp.transpose(sel, (0, 2, 1, 3, 4)).reshape(_B, 3, 48, 48)
    loss = lossp.sum() / (_B * _NQ * _D)
    return loss, sel_img


# speedup vs baseline: 1.8888x; 1.8180x over previous
import numpy as np
import jax
import jax.numpy as jnp
from jax import lax
from jax.experimental import pallas as pl
from jax.experimental.pallas import tpu as pltpu
from jax.experimental.pallas import tpu_sc as plsc

_P = 3
_D = 27
_DP = 32
_B = 4
_NQ = 256
_G = 1468
_GP = 1472
_NGRP = _GP // 16
_NW = 32
_QPW = (_B * _NQ) // _NW

_CV = 3 * 48 * 48
_NSRC = 3 * _CV


def _cubic(t):
    a = -0.75
    at = jnp.abs(t)
    w1 = ((a + 2.0) * at - (a + 3.0)) * at * at + 1.0
    w2 = (((at - 5.0) * at + 8.0) * at - 4.0) * a
    return jnp.where(at <= 1.0, w1, jnp.where(at < 2.0, w2, 0.0))


def _resize1d(x, out_size, axis):
    in_size = x.shape[axis]
    o = jnp.arange(out_size, dtype=jnp.float32)
    src = o * ((in_size - 1) / (out_size - 1))
    i0 = jnp.floor(src).astype(jnp.int32)
    ts = src - i0.astype(jnp.float32)
    offs = jnp.arange(-1, 3)
    idx = jnp.clip(i0[:, None] + offs[None, :], 0, in_size - 1)
    w = _cubic(ts[:, None] - offs[None, :].astype(jnp.float32))
    xm = jnp.moveaxis(x, axis, -1)
    g = xm[..., idx]
    res = jnp.sum(g * w, axis=-1)
    return jnp.moveaxis(res, -1, axis)


def _resize(x, scale):
    _, _, h, w = x.shape
    x = _resize1d(x, int(round(h * scale)), 2)
    x = _resize1d(x, int(round(w * scale)), 3)
    return x


def _base_addrs():
    sizes = [(0, 48), (_CV, 24), (2 * _CV, 12)]
    rows = []

    def piece(base, s, i, j):
        n = (s - _P) // _P if (i or j) else s // _P
        for bi in range(n):
            for bj in range(n):
                rows.append(base + (i + bi * _P) * 48 + j + bj * _P)

    for i in range(1, _P):
        for j in range(1, _P):
            for base, s in sizes:
                piece(base, s, i, j)
    for base, s in sizes:
        piece(base, s, 0, 0)
    assert len(rows) == _G
    rows.extend([2 * _CV + 45 * 48 + 45] * (_GP - _G))
    return np.asarray(rows, np.int32)


_OFF = [c * 2304 + pi * 48 + pj
        for c in range(3) for pi in range(_P) for pj in range(_P)]
_BA = _base_addrs()


def _pad16(v):
    v = list(v)
    return [v[:16], v[16:] + [v[26]] * (_DP - _D)]


_OFFS = np.asarray(
    _pad16(_OFF)
    + _pad16([d // 9 for d in range(_D)])
    + _pad16([(d % 9) // 3 for d in range(_D)])
    + _pad16([d % 3 for d in range(_D)]),
    np.int32)


def _sc_nn_kernel(src_hbm, inp_hbm, ba_hbm, offs_hbm,
                  sel_hbm, loss_hbm,
                  src_v, inp_v, ba_v, offs_v,
                  gc_v, t_v, i_v, img_v, loss_v):
    nc = 2
    wid = lax.axis_index("s") * nc + lax.axis_index("c")
    b = wid // (_NQ // _QPW)
    qbase = (wid % (_NQ // _QPW)) * _QPW

    pltpu.sync_copy(src_hbm.at[b], src_v)
    pltpu.sync_copy(inp_hbm.at[b], inp_v)
    pltpu.sync_copy(ba_hbm, ba_v)
    pltpu.sync_copy(offs_hbm, offs_v)

    lanes = lax.iota(jnp.int32, 16)
    dmask = lanes < (_D - 16)
    orows = [offs_v[k, :] for k in range(8)]

    def stage_q(qi, c):
        q = qbase + qi
        av = lax.broadcast(144 * (q // 16) + 3 * (q % 16), (16,))
        ia = av + orows[0]
        ib = av + orows[1]
        t_v[qi, pl.ds(0, 16)] = plsc.load_gather(src_v, [ia])
        t_v[qi, pl.ds(16, 16)] = plsc.load_gather(src_v, [ib])
        i_v[qi, pl.ds(0, 16)] = plsc.load_gather(inp_v, [ia])
        i_v[qi, pl.ds(16, 16)] = jnp.where(
            dmask, plsc.load_gather(inp_v, [ib]), 0.0)
        return c

    lax.fori_loop(0, _QPW, stage_q, 0)

    def stage_g(gi, c):
        base = gi * 16
        av = ba_v[pl.ds(base, 16)]
        for d in range(_D):
            gc_v[d, pl.ds(base, 16)] = plsc.load_gather(src_v, [av + _OFF[d]])
        return c

    lax.fori_loop(0, _NGRP, stage_g, 0)

    def msp_of(qi):
        t_a = t_v[qi, pl.ds(0, 16)]
        t_b = t_v[qi, pl.ds(16, 16)]
        i_a = i_v[qi, pl.ds(0, 16)]
        i_b = i_v[qi, pl.ds(16, 16)]
        m_a = (t_a + i_a) * 0.5
        m_b = (t_b + i_b) * 0.5
        msp = []
        for d in range(_D):
            ms = m_a[d] if d < 16 else m_b[d - 16]
            msp.append(lax.broadcast(ms, (16,)))
        return msp, i_a, i_b

    def select_q(qi, cstar, i_a, i_b, lossacc):
        av = plsc.load_gather(ba_v, [lax.broadcast(cstar, (16,))])
        sela = plsc.load_gather(src_v, [av + orows[0]])
        selb = jnp.where(dmask, plsc.load_gather(src_v, [av + orows[1]]), 0.0)
        rowb = lax.broadcast(3 * (qi // 16), (16,))
        colb = lax.broadcast(3 * (qi % 16), (16,))
        plsc.store_scatter(
            img_v, [orows[2], rowb + orows[4], colb + orows[6]], sela)
        plsc.store_scatter(
            img_v, [orows[3], rowb + orows[5], colb + orows[7]],
            selb, mask=dmask)
        return (lossacc + jnp.abs(i_a - sela)) + jnp.abs(i_b - selb)

    def q_body(qp, lossacc):
        q1 = 2 * qp
        q2 = q1 + 1
        msp1, ia1, ib1 = msp_of(q1)
        msp2, ia2, ib2 = msp_of(q2)

        def g_body(gh, carry):
            mv1, mg1, mv2, mg2 = carry
            for sub in range(2):
                gi = gh * 2 + sub
                a1 = [jnp.zeros((16,), jnp.float32) for _ in range(4)]
                a2 = [jnp.zeros((16,), jnp.float32) for _ in range(4)]
                base = gi * 16
                for d in range(_D):
                    gv = gc_v[d, pl.ds(base, 16)]
                    df1 = msp1[d] - gv
                    a1[d % 4] = a1[d % 4] + df1 * df1
                    df2 = msp2[d] - gv
                    a2[d % 4] = a2[d % 4] + df2 * df2
                s1 = (a1[0] + a1[1]) + (a1[2] + a1[3])
                s2 = (a2[0] + a2[1]) + (a2[2] + a2[3])
                gb = lax.broadcast(gi, (16,))
                b1 = s1 < mv1
                mv1 = jnp.where(b1, s1, mv1)
                mg1 = jnp.where(b1, gb, mg1)
                b2 = s2 < mv2
                mv2 = jnp.where(b2, s2, mv2)
                mg2 = jnp.where(b2, gb, mg2)
            return mv1, mg1, mv2, mg2

        mv0 = jnp.full((16,), 3.0e38, jnp.float32)
        mg0 = jnp.zeros((16,), jnp.int32)
        mv1, mg1, mv2, mg2 = lax.fori_loop(
            0, _NGRP // 2, g_body, (mv0, mg0, mv0, mg0))

        def argmin_of(mv, mg):
            gmin = jnp.min(mv)
            cand = mg * 16 + lanes
            masked = jnp.where(mv == gmin, cand, jnp.int32(2**30))
            return jnp.min(masked)

        lossacc = select_q(q1, argmin_of(mv1, mg1), ia1, ib1, lossacc)
        return select_q(q2, argmin_of(mv2, mg2), ia2, ib2, lossacc)

    lossacc = lax.fori_loop(0, _QPW // 2, q_body,
                            jnp.zeros((16,), jnp.float32))
    loss_v[...] = lossacc
    pltpu.sync_copy(img_v, sel_hbm.at[b, wid % (_NQ // _QPW)])
    pltpu.sync_copy(loss_v, loss_hbm.at[wid])


def kernel(inp, tar):
    x2 = jnp.pad(_resize(tar, 0.5), ((0, 0), (0, 0), (0, 24), (0, 24)),
                 constant_values=1e9)
    x4 = jnp.pad(_resize(tar, 0.25), ((0, 0), (0, 0), (0, 36), (0, 36)),
                 constant_values=1e9)
    src = jnp.concatenate(
        [tar.reshape(_B, -1), x2.reshape(_B, -1), x4.reshape(_B, -1)],
        axis=1)
    inpf = inp.reshape(_B, -1)

    mesh = plsc.VectorSubcoreMesh(core_axis_name="c", subcore_axis_name="s")
    sel, lossp = pl.kernel(
        _sc_nn_kernel,
        mesh=mesh,
        compiler_params=pltpu.CompilerParams(
            needs_layout_passes=False, use_tc_tiling_on_sc=False),
        out_type=[
            jax.ShapeDtypeStruct((_B, 8, 3, 6, 48), jnp.float32),
            jax.ShapeDtypeStruct((_NW, 16), jnp.float32),
        ],
        scratch_types=[
            pltpu.VMEM((_NSRC,), jnp.float32),
            pltpu.VMEM((_CV,), jnp.float32),
            pltpu.VMEM((_GP,), jnp.int32),
            pltpu.VMEM((8, 16), jnp.int32),
            pltpu.VMEM((_D, _GP), jnp.float32),
            pltpu.VMEM((_QPW, _DP), jnp.float32),
            pltpu.VMEM((_QPW, _DP), jnp.float32),
            pltpu.VMEM((3, 6, 48), jnp.float32),
            pltpu.VMEM((16,), jnp.float32),
        ],
    )(src, inpf, jnp.asarray(_BA), jnp.asarray(_OFFS))

    sel_img = jnp.transpose(sel, (0, 2, 1, 3, 4)).reshape(_B, 3, 48, 48)
    loss = lossp.sum() / (_B * _NQ * _D)
    return loss, sel_img
